# Initial kernel scaffold; baseline (speedup 1.0000x reference)
import jax
import jax.numpy as jnp
from jax.experimental import pallas as pl


def kernel(x, table):
    raise NotImplementedError("write your pallas kernel here")


# speedup vs baseline: 1.9773x; 1.9773x over previous
import functools

import jax
import jax.numpy as jnp
from jax import lax
from jax.experimental import pallas as pl
from jax.experimental.pallas import tpu as pltpu
from jax.experimental.pallas import tpu_sc as plsc

VOCAB = 8192
D = 8192
B = 8192
NW = 32
B_PER_W = B // NW
CH = 4
NCHUNK = B_PER_W // CH
NPAIR = NCHUNK // 2


def _gather_body(idx_hbm, table_hbm, out_hbm, idx_v, rows_v, g0, g1, s0, s1):
    cid = lax.axis_index("c")
    sid = lax.axis_index("s")
    wid = sid * 2 + cid
    base = wid * B_PER_W

    pltpu.sync_copy(idx_hbm.at[wid], idx_v)

    def gather(c, buf, sem):
        return pltpu.make_async_copy(
            table_hbm.at[idx_v.at[c]], rows_v.at[buf], sem)

    def scatter(c, buf, sem):
        return pltpu.make_async_copy(
            rows_v.at[buf], out_hbm.at[pl.ds(base + c * CH, CH)], sem)

    gather(0, 0, g0).start()

    def pair(i, carry):
        c0 = 2 * i
        c1 = c0 + 1

        @pl.when(i > 0)
        def _():
            scatter(c0 - 1, 1, s1).wait()

        gather(c1, 1, g1).start()

        gather(c0, 0, g0).wait()
        scatter(c0, 0, s0).start()

        @pl.when(i < NPAIR - 1)
        def _():
            scatter(c0, 0, s0).wait()
            gather(c0 + 2, 0, g0).start()

        gather(c1, 1, g1).wait()
        scatter(c1, 1, s1).start()
        return carry

    lax.fori_loop(0, NPAIR, pair, 0)

    scatter(NCHUNK - 2, 0, s0).wait()
    scatter(NCHUNK - 1, 1, s1).wait()


@jax.jit
def kernel(x, table):
    idx = x.reshape(NW, NCHUNK, CH).astype(jnp.int32)
    mesh = plsc.VectorSubcoreMesh(core_axis_name="c", subcore_axis_name="s")
    out = pl.kernel(
        _gather_body,
        mesh=mesh,
        out_type=jax.ShapeDtypeStruct((B, D), jnp.float32),
        scratch_types=[
            pltpu.VMEM((NCHUNK, CH), jnp.int32),
            pltpu.VMEM((2, CH, D), jnp.float32),
            pltpu.SemaphoreType.DMA,
            pltpu.SemaphoreType.DMA,
            pltpu.SemaphoreType.DMA,
            pltpu.SemaphoreType.DMA,
        ],
    )(idx, table)
    return out.reshape(x.shape[0], x.shape[1], D)
